# baseline (speedup 1.0000x reference)
import functools

import jax
import jax.numpy as jnp
from jax import lax
from jax.experimental import pallas as pl
from jax.experimental.pallas import tpu as pltpu
from jax.experimental.pallas import tpu_sc as plsc

N = 10000
E = 320000
D = 128
DE = 16

NC = 2
NS = 16
NW = NC * NS
EW = E // NW
B = 128
E_PAD = 327680
EWP = E_PAD // NW
NCHUNK = EWP // B
IG = 16
NG = NCHUNK // IG
N_PAD = 10240
NPT = N_PAD // NS
NBW = N_PAD // NW


def _seg_body(x_hbm, row3, col3, zrows, zvec, onesb, aggp, degp, winp,
              rowc, colc, rows_v, win_v, ones_v, agg_sh, deg_sh,
              sem0, sem1, semd):
    c = lax.axis_index("c")
    s = lax.axis_index("s")
    w = c * NS + s

    mo = jnp.full((16,), -1, jnp.int32)
    lanes = lax.iota(jnp.int32, 16)

    pltpu.sync_copy(zrows, agg_sh.at[pl.ds(s * NPT, NPT)])
    pltpu.sync_copy(zvec, deg_sh.at[pl.ds(s * NPT, NPT)])
    pltpu.sync_copy(onesb, ones_v)

    def _z2(i, carry):
        win_v[pl.ds(i * 16, 16)] = mo
        return carry

    lax.fori_loop(0, N_PAD // 16, _z2, 0)

    plsc.subcore_barrier()

    def _start(j, buf, semb):
        pltpu.async_copy(x_hbm.at[rowc.at[j]], rows_v.at[buf], semb)

    def _finish(g, j, buf, semb):
        pltpu.make_async_copy(x_hbm.at[rowc.at[j]], rows_v.at[buf],
                              semb).wait()
        if j > 0:
            pltpu.make_async_copy(ones_v, deg_sh.at[colc.at[j - 1]],
                                  semd).wait()
        pltpu.async_copy(ones_v, deg_sh.at[colc.at[j]], semd, add=True)
        pltpu.sync_copy(rows_v.at[buf], agg_sh.at[colc.at[j]], add=True)
        base = w * EWP + g * (IG * B) + j * B
        for k in range(B // 16):
            idx = colc[j, pl.ds(k * 16, 16)]
            eids = (base + k * 16) + lanes
            plsc.store_scatter(win_v, (idx,), eids)

    def _group(g, carry):
        pltpu.sync_copy(row3.at[w, g], rowc)
        pltpu.sync_copy(col3.at[w, g], colc)
        _start(0, 0, sem0)
        _start(1, 1, sem1)
        for j in range(IG):
            buf = j % 2
            semb = sem0 if buf == 0 else sem1
            _finish(g, j, buf, semb)
            if j + 2 < IG:
                _start(j + 2, buf, semb)
        pltpu.make_async_copy(ones_v, deg_sh.at[colc.at[IG - 1]],
                              semd).wait()
        return carry

    lax.fori_loop(0, NG, _group, 0)

    pltpu.sync_copy(win_v, winp.at[pl.ds(w * N_PAD, N_PAD)])

    plsc.subcore_barrier()
    pltpu.sync_copy(agg_sh.at[pl.ds(s * NPT, NPT)],
                    aggp.at[c, pl.ds(s * NPT, NPT)])
    pltpu.sync_copy(deg_sh.at[pl.ds(s * NPT, NPT)],
                    degp.at[pl.ds(c * N_PAD + s * NPT, NPT)])


def _combine_body(winp, ea_hbm, wmask, ea,
                  wbuf, idx_v, wm_v, ea_v, sem):
    c = lax.axis_index("c")
    s = lax.axis_index("s")
    w = c * NS + s
    nb = w * NBW

    def _load(t, carry):
        pltpu.async_copy(winp.at[pl.ds(t * N_PAD + nb, NBW)],
                         wbuf.at[pl.ds(t * NBW, NBW)], sem)
        return carry

    lax.fori_loop(0, NW, _load, 0)

    def _drain(t, carry):
        pltpu.make_async_copy(winp.at[pl.ds(t * N_PAD + nb, NBW)],
                              wbuf.at[pl.ds(t * NBW, NBW)], sem).wait()
        return carry

    lax.fori_loop(0, NW, _drain, 0)

    mo = jnp.full((16,), -1, jnp.int32)
    zi = jnp.zeros((16,), jnp.int32)

    for k in range(NBW // 16):
        def _red(t, m):
            return jnp.maximum(m, wbuf[pl.ds(t * NBW + k * 16, 16)])

        m = lax.fori_loop(0, NW, _red, mo)
        wm_v[pl.ds(k * 16, 16)] = jnp.where(m >= 0, 1.0, 0.0)
        idx_v[pl.ds(k * 16, 16)] = jnp.minimum(
            jnp.maximum(m, zi), jnp.full((16,), E - 1, jnp.int32))

    pltpu.async_copy(ea_hbm.at[idx_v], ea_v, sem).wait()

    pltpu.sync_copy(wm_v, wmask.at[pl.ds(nb, NBW)])
    pltpu.sync_copy(ea_v, ea.at[pl.ds(nb, NBW)])


def _dense_body(agg0_ref, agg1_ref, deg0_ref, deg1_ref, x_ref, ea_ref,
                wmask_ref, wl_ref, wr_ref, we_ref, wg1_ref, wg2_ref,
                bl_ref, be_ref, bg_ref, gamma_ref, beta_ref,
                out_ref):
    deg = jnp.maximum(deg0_ref[...] + deg1_ref[...], 1.0)
    agg = (agg0_ref[...] + agg1_ref[...]) / deg
    x = x_ref[...]
    out = (jnp.dot(agg, wl_ref[...], preferred_element_type=jnp.float32)
           + jnp.dot(x, wr_ref[...], preferred_element_type=jnp.float32)
           + bl_ref[...])
    eat = jnp.dot(ea_ref[...], we_ref[...],
                  preferred_element_type=jnp.float32) + be_ref[...]
    pre = (jnp.dot(out, wg1_ref[...], preferred_element_type=jnp.float32)
           + jnp.dot(eat, wg2_ref[...], preferred_element_type=jnp.float32)
           + bg_ref[...])
    gate = 1.0 / (1.0 + jnp.exp(-pre))
    out2 = out + wmask_ref[...] * gate * eat
    mean = jnp.mean(out2, axis=0, keepdims=True)
    cent = out2 - mean
    var = jnp.mean(cent * cent, axis=0, keepdims=True)
    y = cent * jax.lax.rsqrt(var + 1e-5) * gamma_ref[...] + beta_ref[...]
    out_ref[...] = jnp.maximum(2.0 * y, 0.0)


_seg_call = pl.kernel(
    _seg_body,
    out_type=[
        jax.ShapeDtypeStruct((NC, N_PAD, D), jnp.float32),
        jax.ShapeDtypeStruct((NC * N_PAD,), jnp.float32),
        jax.ShapeDtypeStruct((NW * N_PAD,), jnp.int32),
    ],
    mesh=plsc.VectorSubcoreMesh(core_axis_name="c", subcore_axis_name="s"),
    compiler_params=pltpu.CompilerParams(needs_layout_passes=False),
    scratch_types=[
        pltpu.VMEM((IG, B), jnp.int32),
        pltpu.VMEM((IG, B), jnp.int32),
        pltpu.VMEM((2, B, D), jnp.float32),
        pltpu.VMEM((N_PAD,), jnp.int32),
        pltpu.VMEM((B,), jnp.float32),
        pltpu.VMEM_SHARED((N_PAD, D), jnp.float32),
        pltpu.VMEM_SHARED((N_PAD,), jnp.float32),
        pltpu.SemaphoreType.DMA,
        pltpu.SemaphoreType.DMA,
        pltpu.SemaphoreType.DMA,
    ],
)

_combine_call = pl.kernel(
    _combine_body,
    out_type=[
        jax.ShapeDtypeStruct((N_PAD,), jnp.float32),
        jax.ShapeDtypeStruct((N_PAD, DE), jnp.float32),
    ],
    mesh=plsc.VectorSubcoreMesh(core_axis_name="c", subcore_axis_name="s"),
    compiler_params=pltpu.CompilerParams(needs_layout_passes=False,
                                         use_tc_tiling_on_sc=False),
    scratch_types=[
        pltpu.VMEM((NW * NBW,), jnp.int32),
        pltpu.VMEM((NBW,), jnp.int32),
        pltpu.VMEM((NBW,), jnp.float32),
        pltpu.VMEM((NBW, DE), jnp.float32),
        pltpu.SemaphoreType.DMA,
    ],
)


def kernel(x, edge_index, edge_attr, W_l, b_l, W_r, W_e, b_e, W_g, b_g,
           gamma, beta):
    pad = jnp.zeros((E_PAD - E,), jnp.int32)
    padc = N + (lax.iota(jnp.int32, E_PAD - E) % (N_PAD - N))
    rowp = jnp.concatenate([edge_index[0], pad])
    colp = jnp.concatenate([edge_index[1], padc])
    row3 = rowp.reshape(NW, NG, IG, B)
    col3 = colp.reshape(NW, NG, IG, B)

    zrows = jnp.zeros((NPT, D), jnp.float32)
    zvec = jnp.zeros((NPT,), jnp.float32)
    onesb = jnp.ones((B,), jnp.float32)
    aggp, degp, winp = _seg_call(x, row3, col3, zrows, zvec, onesb)
    wmask, ea = _combine_call(winp, edge_attr)

    out = pl.pallas_call(
        _dense_body,
        out_shape=jax.ShapeDtypeStruct((N, D), jnp.float32),
    )(aggp[0, :N], aggp[1, :N], degp[:N, None], degp[N_PAD:N_PAD + N, None],
      x, ea[:N], wmask[:N, None],
      W_l, W_r, W_e, W_g[:D], W_g[D:],
      b_l[None, :], b_e[None, :], b_g[None, :],
      gamma[None, :], beta[None, :])
    return out


# speedup vs baseline: 2.6035x; 2.6035x over previous
import functools

import jax
import jax.numpy as jnp
from jax import lax
from jax.experimental import pallas as pl
from jax.experimental.pallas import tpu as pltpu
from jax.experimental.pallas import tpu_sc as plsc

N = 10000
E = 320000
D = 128
DE = 16

NC = 2
NS = 16
NW = NC * NS
EW = E // NW
B = 128
E_PAD = 327680
EWP = E_PAD // NW
NCHUNK = EWP // B
IG = 16
NG = NCHUNK // IG
N_PAD = 10240
NPT = N_PAD // NS
NBW = N_PAD // NW


def _seg_body(x_hbm, row3, col3, zrows, zvec, onesb, aggp, degp, winp,
              rowc, colc, rows_v, win_v, ones_v, agg_sh, deg_sh,
              sem0, sem1, semd):
    c = lax.axis_index("c")
    s = lax.axis_index("s")
    w = c * NS + s

    mo = jnp.full((16,), -1, jnp.int32)
    lanes = lax.iota(jnp.int32, 16)

    pltpu.sync_copy(zrows, agg_sh.at[pl.ds(s * NPT, NPT)])
    pltpu.sync_copy(zvec, deg_sh.at[pl.ds(s * NPT, NPT)])
    pltpu.sync_copy(onesb, ones_v)

    def _z2(i, carry):
        win_v[pl.ds(i * 16, 16)] = mo
        return carry

    lax.fori_loop(0, N_PAD // 16, _z2, 0)

    plsc.subcore_barrier()

    def _start(j, buf, semb):
        pltpu.async_copy(x_hbm.at[rowc.at[j]], rows_v.at[buf], semb)

    def _finish(g, j, buf, semb):
        pltpu.make_async_copy(x_hbm.at[rowc.at[j]], rows_v.at[buf],
                              semb).wait()
        if j > 0:
            pltpu.make_async_copy(ones_v, deg_sh.at[colc.at[j - 1]],
                                  semd).wait()
        pltpu.async_copy(ones_v, deg_sh.at[colc.at[j]], semd, add=True)
        pltpu.sync_copy(rows_v.at[buf], agg_sh.at[colc.at[j]], add=True)
        base = w * EWP + g * (IG * B) + j * B
        for k in range(B // 16):
            idx = colc[j, pl.ds(k * 16, 16)]
            eids = (base + k * 16) + lanes
            plsc.store_scatter(win_v, (idx,), eids)

    def _group(g, carry):
        pltpu.sync_copy(row3.at[w, g], rowc)
        pltpu.sync_copy(col3.at[w, g], colc)
        _start(0, 0, sem0)
        _start(1, 1, sem1)
        for j in range(IG):
            buf = j % 2
            semb = sem0 if buf == 0 else sem1
            _finish(g, j, buf, semb)
            if j + 2 < IG:
                _start(j + 2, buf, semb)
        pltpu.make_async_copy(ones_v, deg_sh.at[colc.at[IG - 1]],
                              semd).wait()
        return carry

    lax.fori_loop(0, NG, _group, 0)

    pltpu.sync_copy(win_v, winp.at[pl.ds(w * N_PAD, N_PAD)])

    plsc.subcore_barrier()
    pltpu.sync_copy(agg_sh.at[pl.ds(s * NPT, NPT)],
                    aggp.at[c, pl.ds(s * NPT, NPT)])
    pltpu.sync_copy(deg_sh.at[pl.ds(s * NPT, NPT)],
                    degp.at[pl.ds(c * N_PAD + s * NPT, NPT)])


def _combine_body(winp, ea_hbm, wmask, ea,
                  wbuf, idx_v, wm_v, ea_v, sem):
    c = lax.axis_index("c")
    s = lax.axis_index("s")
    w = c * NS + s
    nb = w * NBW

    def _load(t, carry):
        pltpu.async_copy(winp.at[pl.ds(t * N_PAD + nb, NBW)],
                         wbuf.at[pl.ds(t * NBW, NBW)], sem)
        return carry

    lax.fori_loop(0, NW, _load, 0)

    def _drain(t, carry):
        pltpu.make_async_copy(winp.at[pl.ds(t * N_PAD + nb, NBW)],
                              wbuf.at[pl.ds(t * NBW, NBW)], sem).wait()
        return carry

    lax.fori_loop(0, NW, _drain, 0)

    mo = jnp.full((16,), -1, jnp.int32)
    zi = jnp.zeros((16,), jnp.int32)

    for k in range(NBW // 16):
        def _red(t, m):
            return jnp.maximum(m, wbuf[pl.ds(t * NBW + k * 16, 16)])

        m = lax.fori_loop(0, NW, _red, mo)
        wm_v[pl.ds(k * 16, 16)] = jnp.where(m >= 0, 1.0, 0.0)
        idx_v[pl.ds(k * 16, 16)] = jnp.minimum(
            jnp.maximum(m, zi), jnp.full((16,), E - 1, jnp.int32))

    pltpu.async_copy(ea_hbm.at[idx_v], ea_v, sem).wait()

    pltpu.sync_copy(wm_v, wmask.at[pl.ds(nb, NBW)])
    pltpu.sync_copy(ea_v, ea.at[pl.ds(nb, NBW)])


def _dense_body(agg0_ref, agg1_ref, deg0_ref, deg1_ref, x_ref, ea_ref,
                wmask_ref, wl_ref, wr_ref, we_ref, wg1_ref, wg2_ref,
                bl_ref, be_ref, bg_ref, gamma_ref, beta_ref,
                out_ref):
    deg = jnp.maximum(deg0_ref[...] + deg1_ref[...], 1.0)
    agg = (agg0_ref[...] + agg1_ref[...]) / deg
    x = x_ref[...]
    out = (jnp.dot(agg, wl_ref[...], preferred_element_type=jnp.float32)
           + jnp.dot(x, wr_ref[...], preferred_element_type=jnp.float32)
           + bl_ref[...])
    eat = jnp.dot(ea_ref[...], we_ref[...],
                  preferred_element_type=jnp.float32) + be_ref[...]
    pre = (jnp.dot(out, wg1_ref[...], preferred_element_type=jnp.float32)
           + jnp.dot(eat, wg2_ref[...], preferred_element_type=jnp.float32)
           + bg_ref[...])
    gate = 1.0 / (1.0 + jnp.exp(-pre))
    out2 = out + wmask_ref[...] * gate * eat
    mean = jnp.mean(out2, axis=0, keepdims=True)
    cent = out2 - mean
    var = jnp.mean(cent * cent, axis=0, keepdims=True)
    y = cent * jax.lax.rsqrt(var + 1e-5) * gamma_ref[...] + beta_ref[...]
    out_ref[...] = jnp.maximum(2.0 * y, 0.0)


_seg_call = pl.kernel(
    _seg_body,
    out_type=[
        jax.ShapeDtypeStruct((NC, N_PAD, D), jnp.float32),
        jax.ShapeDtypeStruct((NC * N_PAD,), jnp.float32),
        jax.ShapeDtypeStruct((NW * N_PAD,), jnp.int32),
    ],
    mesh=plsc.VectorSubcoreMesh(core_axis_name="c", subcore_axis_name="s"),
    compiler_params=pltpu.CompilerParams(needs_layout_passes=False),
    scratch_types=[
        pltpu.VMEM((IG, B), jnp.int32),
        pltpu.VMEM((IG, B), jnp.int32),
        pltpu.VMEM((2, B, D), jnp.float32),
        pltpu.VMEM((N_PAD,), jnp.int32),
        pltpu.VMEM((B,), jnp.float32),
        pltpu.VMEM_SHARED((N_PAD, D), jnp.float32),
        pltpu.VMEM_SHARED((N_PAD,), jnp.float32),
        pltpu.SemaphoreType.DMA,
        pltpu.SemaphoreType.DMA,
        pltpu.SemaphoreType.DMA,
    ],
)

_combine_call = pl.kernel(
    _combine_body,
    out_type=[
        jax.ShapeDtypeStruct((N_PAD,), jnp.float32),
        jax.ShapeDtypeStruct((N_PAD, DE), jnp.float32),
    ],
    mesh=plsc.VectorSubcoreMesh(core_axis_name="c", subcore_axis_name="s"),
    compiler_params=pltpu.CompilerParams(needs_layout_passes=False,
                                         use_tc_tiling_on_sc=False),
    scratch_types=[
        pltpu.VMEM((NW * NBW,), jnp.int32),
        pltpu.VMEM((NBW,), jnp.int32),
        pltpu.VMEM((NBW,), jnp.float32),
        pltpu.VMEM((NBW, DE), jnp.float32),
        pltpu.SemaphoreType.DMA,
    ],
)


def kernel(x, edge_index, edge_attr, W_l, b_l, W_r, W_e, b_e, W_g, b_g,
           gamma, beta):
    padr = lax.iota(jnp.int32, E_PAD - E) % N
    padc = N + (lax.iota(jnp.int32, E_PAD - E) % (N_PAD - N))
    rowp = jnp.concatenate([edge_index[0], padr])
    colp = jnp.concatenate([edge_index[1], padc])
    row3 = rowp.reshape(NW, NG, IG, B)
    col3 = colp.reshape(NW, NG, IG, B)

    zrows = jnp.zeros((NPT, D), jnp.float32)
    zvec = jnp.zeros((NPT,), jnp.float32)
    onesb = jnp.ones((B,), jnp.float32)
    aggp, degp, winp = _seg_call(x, row3, col3, zrows, zvec, onesb)
    wmask, ea = _combine_call(winp, edge_attr)

    out = pl.pallas_call(
        _dense_body,
        out_shape=jax.ShapeDtypeStruct((N, D), jnp.float32),
    )(aggp[0, :N], aggp[1, :N], degp[:N, None], degp[N_PAD:N_PAD + N, None],
      x, ea[:N], wmask[:N, None],
      W_l, W_r, W_e, W_g[:D], W_g[D:],
      b_l[None, :], b_e[None, :], b_g[None, :],
      gamma[None, :], beta[None, :])
    return out


# speedup vs baseline: 2.7766x; 1.0665x over previous
import functools

import jax
import jax.numpy as jnp
from jax import lax
from jax.experimental import pallas as pl
from jax.experimental.pallas import tpu as pltpu
from jax.experimental.pallas import tpu_sc as plsc

N = 10000
E = 320000
D = 128
DE = 16

NC = 2
NS = 16
NW = NC * NS
EW = E // NW
B = 128
E_PAD = 327680
EWP = E_PAD // NW
NCHUNK = EWP // B
IG = 16
NG = NCHUNK // IG
N_PAD = 10240
NPT = N_PAD // NS
NBW = N_PAD // NW


def _seg_body(x_hbm, row3, col3, zrows, zvec, onesb, aggp, degp, winp,
              rowc, colc, rows_v, win_v, ones_v, agg_sh, deg_sh,
              sem0, sem1, semd):
    c = lax.axis_index("c")
    s = lax.axis_index("s")
    w = c * NS + s

    mo = jnp.full((16,), -1, jnp.int32)
    lanes = lax.iota(jnp.int32, 16)

    pltpu.sync_copy(zrows, agg_sh.at[pl.ds(s * NPT, NPT)])
    pltpu.sync_copy(zvec, deg_sh.at[pl.ds(s * NPT, NPT)])
    pltpu.sync_copy(onesb, ones_v)

    def _z2(i, carry):
        win_v[pl.ds(i * 16, 16)] = mo
        return carry

    lax.fori_loop(0, N_PAD // 16, _z2, 0)

    plsc.subcore_barrier()

    def _start(j, buf, semb):
        pltpu.async_copy(x_hbm.at[rowc.at[j]], rows_v.at[buf], semb)

    def _finish(g, j, buf, semb):
        pltpu.make_async_copy(x_hbm.at[rowc.at[j]], rows_v.at[buf],
                              semb).wait()
        if j > 0:
            pltpu.make_async_copy(ones_v, deg_sh.at[colc.at[j - 1]],
                                  semd).wait()
        pltpu.async_copy(ones_v, deg_sh.at[colc.at[j]], semd, add=True)
        pltpu.sync_copy(rows_v.at[buf], agg_sh.at[colc.at[j]], add=True)
        base = w * EWP + g * (IG * B) + j * B
        for k in range(B // 16):
            idx = colc[j, pl.ds(k * 16, 16)]
            eids = (base + k * 16) + lanes
            plsc.store_scatter(win_v, (idx,), eids)

    def _group(g, carry):
        pltpu.sync_copy(row3.at[w, g], rowc)
        pltpu.sync_copy(col3.at[w, g], colc)
        _start(0, 0, sem0)
        _start(1, 1, sem1)
        for j in range(IG):
            buf = j % 2
            semb = sem0 if buf == 0 else sem1
            _finish(g, j, buf, semb)
            if j + 2 < IG:
                _start(j + 2, buf, semb)
        pltpu.make_async_copy(ones_v, deg_sh.at[colc.at[IG - 1]],
                              semd).wait()
        return carry

    lax.fori_loop(0, NG, _group, 0)

    pltpu.sync_copy(win_v, winp.at[pl.ds(w * N_PAD, N_PAD)])

    plsc.subcore_barrier()
    pltpu.sync_copy(agg_sh.at[pl.ds(s * NPT, NPT)],
                    aggp.at[c, pl.ds(s * NPT, NPT)])
    pltpu.sync_copy(deg_sh.at[pl.ds(s * NPT, NPT)],
                    degp.at[pl.ds(c * N_PAD + s * NPT, NPT)])


def _combine_body(winp, ea_hbm, wmask, ea,
                  wbuf, idx_v, wm_v, ea_v, sem):
    c = lax.axis_index("c")
    s = lax.axis_index("s")
    w = c * NS + s
    nb = w * NBW

    def _load(t, carry):
        pltpu.async_copy(winp.at[pl.ds(t * N_PAD + nb, NBW)],
                         wbuf.at[pl.ds(t * NBW, NBW)], sem)
        return carry

    lax.fori_loop(0, NW, _load, 0)

    def _drain(t, carry):
        pltpu.make_async_copy(winp.at[pl.ds(t * N_PAD + nb, NBW)],
                              wbuf.at[pl.ds(t * NBW, NBW)], sem).wait()
        return carry

    lax.fori_loop(0, NW, _drain, 0)

    mo = jnp.full((16,), -1, jnp.int32)
    zi = jnp.zeros((16,), jnp.int32)

    for k in range(NBW // 16):
        def _red(t, m):
            return jnp.maximum(m, wbuf[pl.ds(t * NBW + k * 16, 16)])

        m = lax.fori_loop(0, NW, _red, mo)
        wm_v[pl.ds(k * 16, 16)] = jnp.where(m >= 0, 1.0, 0.0)
        idx_v[pl.ds(k * 16, 16)] = jnp.minimum(
            jnp.maximum(m, zi), jnp.full((16,), E - 1, jnp.int32))

    pltpu.async_copy(ea_hbm.at[idx_v], ea_v, sem).wait()

    pltpu.sync_copy(wm_v, wmask.at[pl.ds(nb, NBW)])
    pltpu.sync_copy(ea_v, ea.at[pl.ds(nb, NBW)])


def _dense_body(aggp_ref, degp_ref, x_ref, ea_ref,
                wmask_ref, wl_ref, wr_ref, we_ref, wg1_ref, wg2_ref,
                bl_ref, be_ref, bg_ref, gamma_ref, beta_ref,
                out_ref):
    deg = jnp.maximum(degp_ref[0, :N] + degp_ref[1, :N], 1.0)[:, None]
    agg = (aggp_ref[0, :N] + aggp_ref[1, :N]) / deg
    x = x_ref[...]
    out = (jnp.dot(agg, wl_ref[...], preferred_element_type=jnp.float32)
           + jnp.dot(x, wr_ref[...], preferred_element_type=jnp.float32)
           + bl_ref[...])
    eat = jnp.dot(ea_ref[:N], we_ref[...],
                  preferred_element_type=jnp.float32) + be_ref[...]
    pre = (jnp.dot(out, wg1_ref[...], preferred_element_type=jnp.float32)
           + jnp.dot(eat, wg2_ref[...], preferred_element_type=jnp.float32)
           + bg_ref[...])
    gate = 1.0 / (1.0 + jnp.exp(-pre))
    out2 = out + wmask_ref[0, :N, :] * gate * eat
    mean = jnp.mean(out2, axis=0, keepdims=True)
    cent = out2 - mean
    var = jnp.mean(cent * cent, axis=0, keepdims=True)
    y = cent * jax.lax.rsqrt(var + 1e-5) * gamma_ref[...] + beta_ref[...]
    out_ref[...] = jnp.maximum(2.0 * y, 0.0)


_seg_call = pl.kernel(
    _seg_body,
    out_type=[
        jax.ShapeDtypeStruct((NC, N_PAD, D), jnp.float32),
        jax.ShapeDtypeStruct((NC * N_PAD,), jnp.float32),
        jax.ShapeDtypeStruct((NW * N_PAD,), jnp.int32),
    ],
    mesh=plsc.VectorSubcoreMesh(core_axis_name="c", subcore_axis_name="s"),
    compiler_params=pltpu.CompilerParams(needs_layout_passes=False),
    scratch_types=[
        pltpu.VMEM((IG, B), jnp.int32),
        pltpu.VMEM((IG, B), jnp.int32),
        pltpu.VMEM((2, B, D), jnp.float32),
        pltpu.VMEM((N_PAD,), jnp.int32),
        pltpu.VMEM((B,), jnp.float32),
        pltpu.VMEM_SHARED((N_PAD, D), jnp.float32),
        pltpu.VMEM_SHARED((N_PAD,), jnp.float32),
        pltpu.SemaphoreType.DMA,
        pltpu.SemaphoreType.DMA,
        pltpu.SemaphoreType.DMA,
    ],
)

_combine_call = pl.kernel(
    _combine_body,
    out_type=[
        jax.ShapeDtypeStruct((N_PAD,), jnp.float32),
        jax.ShapeDtypeStruct((N_PAD, DE), jnp.float32),
    ],
    mesh=plsc.VectorSubcoreMesh(core_axis_name="c", subcore_axis_name="s"),
    compiler_params=pltpu.CompilerParams(needs_layout_passes=False,
                                         use_tc_tiling_on_sc=False),
    scratch_types=[
        pltpu.VMEM((NW * NBW,), jnp.int32),
        pltpu.VMEM((NBW,), jnp.int32),
        pltpu.VMEM((NBW,), jnp.float32),
        pltpu.VMEM((NBW, DE), jnp.float32),
        pltpu.SemaphoreType.DMA,
    ],
)


def kernel(x, edge_index, edge_attr, W_l, b_l, W_r, W_e, b_e, W_g, b_g,
           gamma, beta):
    padr = lax.iota(jnp.int32, E_PAD - E) % N
    padc = N + (lax.iota(jnp.int32, E_PAD - E) % (N_PAD - N))
    rowp = jnp.concatenate([edge_index[0], padr])
    colp = jnp.concatenate([edge_index[1], padc])
    row3 = rowp.reshape(NW, NG, IG, B)
    col3 = colp.reshape(NW, NG, IG, B)

    zrows = jnp.zeros((NPT, D), jnp.float32)
    zvec = jnp.zeros((NPT,), jnp.float32)
    onesb = jnp.ones((B,), jnp.float32)
    aggp, degp, winp = _seg_call(x, row3, col3, zrows, zvec, onesb)
    wmask, ea = _combine_call(winp, edge_attr)

    out = pl.pallas_call(
        _dense_body,
        out_shape=jax.ShapeDtypeStruct((N, D), jnp.float32),
    )(aggp, degp.reshape(NC, N_PAD), x, ea, wmask.reshape(1, N_PAD, 1),
      W_l, W_r, W_e, W_g[:D], W_g[D:],
      b_l[None, :], b_e[None, :], b_g[None, :],
      gamma[None, :], beta[None, :])
    return out
